# baseline (speedup 1.0000x reference)
import functools
import math

import jax
import jax.numpy as jnp
from jax import lax
from jax.experimental import pallas as pl
from jax.experimental.pallas import tpu as pltpu
from jax.experimental.pallas import tpu_sc as plsc

N = 10000
E = 320000
D = 128
L = 3

NC = 2
NS = 16
NW = NC * NS
EPW = E // NW
CHUNK = 96
NCHUNK = -(-EPW // CHUNK)
EPAD = NCHUNK * CHUNK - EPW
NDUMMY = 8
NRB = 3
NIB = 6
UNIT = 40
NUNITS = N // UNIT

BLK = 1000


def _segment_sum_partials(m, edges4):
    mesh = plsc.VectorSubcoreMesh(core_axis_name="c", subcore_axis_name="s")

    @functools.partial(
        pl.kernel,
        mesh=mesh,
        out_type=jax.ShapeDtypeStruct((NC, N, D), jnp.float32),
        scratch_types=[
            pltpu.VMEM((NIB, 2, CHUNK), jnp.int32),
            pltpu.VMEM((NRB, CHUNK, D), jnp.float32),
            pltpu.VMEM_SHARED((N + NDUMMY, D), jnp.float32),
            [pltpu.SemaphoreType.DMA] * NIB,
            [pltpu.SemaphoreType.DMA] * NRB,
            [pltpu.SemaphoreType.DMA] * NRB,
        ],
    )
    def seg_kernel(m_hbm, edges_hbm, out_hbm,
                   idx_v, rows_v, agg_sh, isems, gsems, ssems):
        c = lax.axis_index("c")
        s = lax.axis_index("s")
        wid = c * NS + s
        nu = jnp.where(s < NUNITS - NS * (NUNITS // NS), NUNITS // NS + 1,
                       NUNITS // NS)

        def zfill(i, carry):
            for g in range(D // 16):
                rows_v[0, i, pl.ds(g * 16, 16)] = jnp.zeros((16,), jnp.float32)
            return carry

        lax.fori_loop(0, UNIT, zfill, 0)

        def zcopy(k, carry):
            pltpu.sync_copy(rows_v.at[0, pl.ds(0, UNIT)],
                            agg_sh.at[pl.ds((s + NS * k) * UNIT, UNIT)])
            return carry

        lax.fori_loop(0, nu, zcopy, 0)

        plsc.subcore_barrier()

        def idx_load(j, q):
            pltpu.async_copy(edges_hbm.at[wid, j], idx_v.at[q], isems[q])

        def idx_wait(j, q):
            pltpu.make_async_copy(edges_hbm.at[wid, j], idx_v.at[q],
                                  isems[q]).wait()

        def gather(q, r):
            pltpu.async_copy(m_hbm.at[idx_v.at[q, 0]], rows_v.at[r],
                             gsems[r])

        def gather_wait(q, r):
            pltpu.make_async_copy(m_hbm.at[idx_v.at[q, 0]], rows_v.at[r],
                                  gsems[r]).wait()

        def scatter(q, r):
            pltpu.async_copy(rows_v.at[r], agg_sh.at[idx_v.at[q, 1]],
                             ssems[r], add=True)

        def scatter_wait(q, r):
            pltpu.make_async_copy(rows_v.at[r], agg_sh.at[idx_v.at[q, 1]],
                                  ssems[r]).wait()

        def step(j, q, r, first=False):
            gather_wait(q, r)
            scatter(q, r)
            if not first:
                scatter_wait((q + NIB - 1) % NIB, (r + NRB - 1) % NRB)

            @pl.when(j + NRB - 1 < NCHUNK)
            def _():
                idx_wait(j + NRB - 1, (q + NRB - 1) % NIB)
                gather((q + NRB - 1) % NIB, (r + NRB - 1) % NRB)

            @pl.when(j + NIB - 2 < NCHUNK)
            def _():
                idx_load(j + NIB - 2, (q + NIB - 2) % NIB)

        for q in range(NIB - 2):
            idx_load(q, q)
        for t in range(NRB - 1):
            idx_wait(t, t)
            gather(t, t)

        JS = NRB - 1
        UNROLL = NRB * NIB // math.gcd(NRB, NIB)
        for j in range(JS):
            step(j, j % NIB, j % NRB, first=(j == 0))

        def main_body(k, carry):
            j0 = JS + UNROLL * k
            for i in range(UNROLL):
                step(j0 + i, (JS + i) % NIB, (JS + i) % NRB)
            return carry

        n_main = (NCHUNK - JS) // UNROLL
        lax.fori_loop(0, n_main, main_body, 0)
        for j in range(JS + UNROLL * n_main, NCHUNK):
            step(j, j % NIB, j % NRB)

        scatter_wait((NCHUNK - 1) % NIB, (NCHUNK - 1) % NRB)

        plsc.subcore_barrier()

        def wcopy(k, carry):
            off = (s + NS * k) * UNIT
            pltpu.sync_copy(agg_sh.at[pl.ds(off, UNIT)],
                            out_hbm.at[c, pl.ds(off, UNIT)])
            return carry

        lax.fori_loop(0, nu, wcopy, 0)

    return seg_kernel(m, edges4)


def _mm_body(x_ref, w_ref, o_ref):
    o_ref[...] = jnp.dot(x_ref[...], w_ref[...],
                         preferred_element_type=jnp.float32)


def _message_mm(x, w):
    return pl.pallas_call(
        _mm_body,
        grid=(N // BLK,),
        in_specs=[
            pl.BlockSpec((BLK, D), lambda i: (i, 0)),
            pl.BlockSpec((D, D), lambda i: (0, 0)),
        ],
        out_specs=pl.BlockSpec((BLK, D), lambda i: (i, 0)),
        out_shape=jax.ShapeDtypeStruct((N, D), jnp.float32),
    )(x, w)


def _gru_body(p_ref, h_ref, wih_ref, whh_ref, bih_ref, bhh_ref, wn_ref,
              hy_ref, mn_ref):
    agg = p_ref[0] + p_ref[1]
    h = h_ref[...]
    gi = jnp.dot(agg, wih_ref[...], preferred_element_type=jnp.float32)
    gi = gi + bih_ref[...]
    gh = jnp.dot(h, whh_ref[...], preferred_element_type=jnp.float32)
    gh = gh + bhh_ref[...]
    r = jax.nn.sigmoid(gi[:, :D] + gh[:, :D])
    z = jax.nn.sigmoid(gi[:, D:2 * D] + gh[:, D:2 * D])
    n = jnp.tanh(gi[:, 2 * D:] + r * gh[:, 2 * D:])
    hy = (1.0 - z) * n + z * h
    hy_ref[...] = hy
    mn_ref[...] = jnp.dot(hy, wn_ref[...], preferred_element_type=jnp.float32)


def _gru_layer(p, h, wih_t, whh_t, bih, bhh, w_next):
    return pl.pallas_call(
        _gru_body,
        grid=(N // BLK,),
        in_specs=[
            pl.BlockSpec((2, BLK, D), lambda i: (0, i, 0)),
            pl.BlockSpec((BLK, D), lambda i: (i, 0)),
            pl.BlockSpec((D, 3 * D), lambda i: (0, 0)),
            pl.BlockSpec((D, 3 * D), lambda i: (0, 0)),
            pl.BlockSpec((1, 3 * D), lambda i: (0, 0)),
            pl.BlockSpec((1, 3 * D), lambda i: (0, 0)),
            pl.BlockSpec((D, D), lambda i: (0, 0)),
        ],
        out_specs=[
            pl.BlockSpec((BLK, D), lambda i: (i, 0)),
            pl.BlockSpec((BLK, D), lambda i: (i, 0)),
        ],
        out_shape=[
            jax.ShapeDtypeStruct((N, D), jnp.float32),
            jax.ShapeDtypeStruct((N, D), jnp.float32),
        ],
    )(p, h, wih_t, whh_t, bih, bhh, w_next)


def kernel(x_encoded, edge_index, mapping_idx, weight, w_ih, w_hh, b_ih, b_hh):
    del mapping_idx
    src2 = jnp.pad(edge_index[0].reshape(NW, EPW), ((0, 0), (0, EPAD)),
                   constant_values=0)
    dst2 = jnp.pad(edge_index[1].reshape(NW, EPW), ((0, 0), (0, EPAD)),
                   constant_values=N)
    edges4 = jnp.stack([src2.reshape(NW, NCHUNK, CHUNK),
                        dst2.reshape(NW, NCHUNK, CHUNK)], axis=2)
    wih_t = w_ih.T
    whh_t = w_hh.T
    bih = b_ih.reshape(1, 3 * D)
    bhh = b_hh.reshape(1, 3 * D)

    h = x_encoded
    m = _message_mm(x_encoded, weight[0])
    for i in range(L):
        p = _segment_sum_partials(m, edges4)
        h, m = _gru_layer(p, h, wih_t, whh_t, bih, bhh, weight[(i + 1) % L])
    return h


# speedup vs baseline: 1.3418x; 1.3418x over previous
import functools
import math

import jax
import jax.numpy as jnp
from jax import lax
from jax.experimental import pallas as pl
from jax.experimental.pallas import tpu as pltpu
from jax.experimental.pallas import tpu_sc as plsc

N = 10000
E = 320000
D = 128
L = 3

NC = 2
NS = 16
NW = NC * NS
EPW = E // NW
CHUNK = 88
NCHUNK = -(-EPW // CHUNK)
EPAD = NCHUNK * CHUNK - EPW
NDUMMY = 8
NRB = 3
NIB = 6
UNIT = 40
NUNITS = N // UNIT

BLK = 1000


def _segment_sum_partials(m, edges4):
    mesh = plsc.VectorSubcoreMesh(core_axis_name="c", subcore_axis_name="s")

    @functools.partial(
        pl.kernel,
        mesh=mesh,
        out_type=jax.ShapeDtypeStruct((NC, N, D), jnp.float32),
        scratch_types=[
            pltpu.VMEM((NIB, 2, CHUNK), jnp.int32),
            pltpu.VMEM((NRB, CHUNK, D), jnp.float32),
            pltpu.VMEM_SHARED((N + NDUMMY, D), jnp.float32),
            [pltpu.SemaphoreType.DMA] * NIB,
            [pltpu.SemaphoreType.DMA] * NRB,
            [pltpu.SemaphoreType.DMA] * NRB,
        ],
    )
    def seg_kernel(m_hbm, edges_hbm, out_hbm,
                   idx_v, rows_v, agg_sh, isems, gsems, ssems):
        c = lax.axis_index("c")
        s = lax.axis_index("s")
        wid = c * NS + s
        nu = jnp.where(s < NUNITS - NS * (NUNITS // NS), NUNITS // NS + 1,
                       NUNITS // NS)

        def zfill(i, carry):
            for g in range(D // 16):
                rows_v[0, i, pl.ds(g * 16, 16)] = jnp.zeros((16,), jnp.float32)
            return carry

        lax.fori_loop(0, UNIT, zfill, 0)

        def zcopy(k, carry):
            pltpu.sync_copy(rows_v.at[0, pl.ds(0, UNIT)],
                            agg_sh.at[pl.ds((s + NS * k) * UNIT, UNIT)])
            return carry

        lax.fori_loop(0, nu, zcopy, 0)

        plsc.subcore_barrier()

        def idx_load(j, q):
            pltpu.async_copy(edges_hbm.at[wid, j], idx_v.at[q], isems[q])

        def idx_wait(j, q):
            pltpu.make_async_copy(edges_hbm.at[wid, j], idx_v.at[q],
                                  isems[q]).wait()

        def gather(q, r):
            pltpu.async_copy(m_hbm.at[idx_v.at[q, 0]], rows_v.at[r],
                             gsems[r])

        def gather_wait(q, r):
            pltpu.make_async_copy(m_hbm.at[idx_v.at[q, 0]], rows_v.at[r],
                                  gsems[r]).wait()

        def scatter(q, r):
            pltpu.async_copy(rows_v.at[r], agg_sh.at[idx_v.at[q, 1]],
                             ssems[r], add=True)

        def scatter_wait(q, r):
            pltpu.make_async_copy(rows_v.at[r], agg_sh.at[idx_v.at[q, 1]],
                                  ssems[r]).wait()

        def step(j, q, r, first=False):
            gather_wait(q, r)
            scatter(q, r)
            if not first:
                scatter_wait((q + NIB - 1) % NIB, (r + NRB - 1) % NRB)

            @pl.when(j + NRB - 1 < NCHUNK)
            def _():
                idx_wait(j + NRB - 1, (q + NRB - 1) % NIB)
                gather((q + NRB - 1) % NIB, (r + NRB - 1) % NRB)

            @pl.when(j + NIB - 2 < NCHUNK)
            def _():
                idx_load(j + NIB - 2, (q + NIB - 2) % NIB)

        for q in range(NIB - 2):
            idx_load(q, q)
        for t in range(NRB - 1):
            idx_wait(t, t)
            gather(t, t)

        JS = NRB - 1
        UNROLL = NRB * NIB // math.gcd(NRB, NIB)
        for j in range(JS):
            step(j, j % NIB, j % NRB, first=(j == 0))

        def main_body(k, carry):
            j0 = JS + UNROLL * k
            for i in range(UNROLL):
                step(j0 + i, (JS + i) % NIB, (JS + i) % NRB)
            return carry

        n_main = (NCHUNK - JS) // UNROLL
        lax.fori_loop(0, n_main, main_body, 0)
        for j in range(JS + UNROLL * n_main, NCHUNK):
            step(j, j % NIB, j % NRB)

        scatter_wait((NCHUNK - 1) % NIB, (NCHUNK - 1) % NRB)

        plsc.subcore_barrier()

        def wcopy(k, carry):
            off = (s + NS * k) * UNIT
            pltpu.sync_copy(agg_sh.at[pl.ds(off, UNIT)],
                            out_hbm.at[c, pl.ds(off, UNIT)])
            return carry

        lax.fori_loop(0, nu, wcopy, 0)

    return seg_kernel(m, edges4)


def _mm_body(x_ref, w_ref, o_ref):
    o_ref[...] = jnp.dot(x_ref[...], w_ref[...],
                         preferred_element_type=jnp.float32)


def _message_mm(x, w):
    return pl.pallas_call(
        _mm_body,
        grid=(N // BLK,),
        in_specs=[
            pl.BlockSpec((BLK, D), lambda i: (i, 0)),
            pl.BlockSpec((D, D), lambda i: (0, 0)),
        ],
        out_specs=pl.BlockSpec((BLK, D), lambda i: (i, 0)),
        out_shape=jax.ShapeDtypeStruct((N, D), jnp.float32),
    )(x, w)


def _gru_body(p_ref, h_ref, wih_ref, whh_ref, bih_ref, bhh_ref, wn_ref,
              hy_ref, mn_ref):
    agg = p_ref[0] + p_ref[1]
    h = h_ref[...]
    gi = jnp.dot(agg, wih_ref[...], preferred_element_type=jnp.float32)
    gi = gi + bih_ref[...]
    gh = jnp.dot(h, whh_ref[...], preferred_element_type=jnp.float32)
    gh = gh + bhh_ref[...]
    r = jax.nn.sigmoid(gi[:, :D] + gh[:, :D])
    z = jax.nn.sigmoid(gi[:, D:2 * D] + gh[:, D:2 * D])
    n = jnp.tanh(gi[:, 2 * D:] + r * gh[:, 2 * D:])
    hy = (1.0 - z) * n + z * h
    hy_ref[...] = hy
    mn_ref[...] = jnp.dot(hy, wn_ref[...], preferred_element_type=jnp.float32)


def _gru_layer(p, h, wih_t, whh_t, bih, bhh, w_next):
    return pl.pallas_call(
        _gru_body,
        grid=(N // BLK,),
        in_specs=[
            pl.BlockSpec((2, BLK, D), lambda i: (0, i, 0)),
            pl.BlockSpec((BLK, D), lambda i: (i, 0)),
            pl.BlockSpec((D, 3 * D), lambda i: (0, 0)),
            pl.BlockSpec((D, 3 * D), lambda i: (0, 0)),
            pl.BlockSpec((1, 3 * D), lambda i: (0, 0)),
            pl.BlockSpec((1, 3 * D), lambda i: (0, 0)),
            pl.BlockSpec((D, D), lambda i: (0, 0)),
        ],
        out_specs=[
            pl.BlockSpec((BLK, D), lambda i: (i, 0)),
            pl.BlockSpec((BLK, D), lambda i: (i, 0)),
        ],
        out_shape=[
            jax.ShapeDtypeStruct((N, D), jnp.float32),
            jax.ShapeDtypeStruct((N, D), jnp.float32),
        ],
    )(p, h, wih_t, whh_t, bih, bhh, w_next)


def kernel(x_encoded, edge_index, mapping_idx, weight, w_ih, w_hh, b_ih, b_hh):
    del mapping_idx
    src2 = jnp.pad(edge_index[0].reshape(NW, EPW), ((0, 0), (0, EPAD)),
                   constant_values=0)
    dst2 = jnp.pad(edge_index[1].reshape(NW, EPW), ((0, 0), (0, EPAD)),
                   constant_values=N)
    edges4 = jnp.stack([src2.reshape(NW, NCHUNK, CHUNK),
                        dst2.reshape(NW, NCHUNK, CHUNK)], axis=2)
    wih_t = w_ih.T
    whh_t = w_hh.T
    bih = b_ih.reshape(1, 3 * D)
    bhh = b_hh.reshape(1, 3 * D)

    h = x_encoded
    m = _message_mm(x_encoded, weight[0])
    for i in range(L):
        p = _segment_sum_partials(m, edges4)
        h, m = _gru_layer(p, h, wih_t, whh_t, bih, bhh, weight[(i + 1) % L])
    return h


# speedup vs baseline: 1.6654x; 1.2412x over previous
import functools
import math

import jax
import jax.numpy as jnp
from jax import lax
from jax.experimental import pallas as pl
from jax.experimental.pallas import tpu as pltpu
from jax.experimental.pallas import tpu_sc as plsc

N = 10000
E = 320000
D = 128
L = 3

NC = 2
NS = 16
NW = NC * NS
EPW = E // NW
CHUNK = 80
NCHUNK = -(-EPW // CHUNK)
EPAD = NCHUNK * CHUNK - EPW
NDUMMY = 8
NRB = 3
NIB = 6
UNIT = 40
NUNITS = N // UNIT

BLK = 1000


def _segment_sum_partials(m, edges4):
    mesh = plsc.VectorSubcoreMesh(core_axis_name="c", subcore_axis_name="s")

    @functools.partial(
        pl.kernel,
        mesh=mesh,
        out_type=jax.ShapeDtypeStruct((NC, N, D), jnp.float32),
        scratch_types=[
            pltpu.VMEM((NIB, 2, CHUNK), jnp.int32),
            pltpu.VMEM((NRB, CHUNK, D), jnp.float32),
            pltpu.VMEM_SHARED((N + NDUMMY, D), jnp.float32),
            [pltpu.SemaphoreType.DMA] * NIB,
            [pltpu.SemaphoreType.DMA] * NRB,
            [pltpu.SemaphoreType.DMA] * NRB,
        ],
    )
    def seg_kernel(m_hbm, edges_hbm, out_hbm,
                   idx_v, rows_v, agg_sh, isems, gsems, ssems):
        c = lax.axis_index("c")
        s = lax.axis_index("s")
        wid = c * NS + s
        nu = jnp.where(s < NUNITS - NS * (NUNITS // NS), NUNITS // NS + 1,
                       NUNITS // NS)

        def zfill(i, carry):
            for g in range(D // 16):
                rows_v[0, i, pl.ds(g * 16, 16)] = jnp.zeros((16,), jnp.float32)
            return carry

        lax.fori_loop(0, UNIT, zfill, 0)

        def zcopy(k, carry):
            pltpu.sync_copy(rows_v.at[0, pl.ds(0, UNIT)],
                            agg_sh.at[pl.ds((s + NS * k) * UNIT, UNIT)])
            return carry

        lax.fori_loop(0, nu, zcopy, 0)

        plsc.subcore_barrier()

        def idx_load(j, q):
            pltpu.async_copy(edges_hbm.at[wid, j], idx_v.at[q], isems[q])

        def idx_wait(j, q):
            pltpu.make_async_copy(edges_hbm.at[wid, j], idx_v.at[q],
                                  isems[q]).wait()

        def gather(q, r):
            pltpu.async_copy(m_hbm.at[idx_v.at[q, 0]], rows_v.at[r],
                             gsems[r])

        def gather_wait(q, r):
            pltpu.make_async_copy(m_hbm.at[idx_v.at[q, 0]], rows_v.at[r],
                                  gsems[r]).wait()

        def scatter(q, r):
            pltpu.async_copy(rows_v.at[r], agg_sh.at[idx_v.at[q, 1]],
                             ssems[r], add=True)

        def scatter_wait(q, r):
            pltpu.make_async_copy(rows_v.at[r], agg_sh.at[idx_v.at[q, 1]],
                                  ssems[r]).wait()

        def step(j, q, r, first=False):
            gather_wait(q, r)
            scatter(q, r)
            if not first:
                scatter_wait((q + NIB - 1) % NIB, (r + NRB - 1) % NRB)

            @pl.when(j + NRB - 1 < NCHUNK)
            def _():
                idx_wait(j + NRB - 1, (q + NRB - 1) % NIB)
                gather((q + NRB - 1) % NIB, (r + NRB - 1) % NRB)

            @pl.when(j + NIB - 2 < NCHUNK)
            def _():
                idx_load(j + NIB - 2, (q + NIB - 2) % NIB)

        for q in range(NIB - 2):
            idx_load(q, q)
        for t in range(NRB - 1):
            idx_wait(t, t)
            gather(t, t)

        JS = NRB - 1
        UNROLL = NRB * NIB // math.gcd(NRB, NIB)
        for j in range(JS):
            step(j, j % NIB, j % NRB, first=(j == 0))

        def main_body(k, carry):
            j0 = JS + UNROLL * k
            for i in range(UNROLL):
                step(j0 + i, (JS + i) % NIB, (JS + i) % NRB)
            return carry

        n_main = (NCHUNK - JS) // UNROLL
        lax.fori_loop(0, n_main, main_body, 0)
        for j in range(JS + UNROLL * n_main, NCHUNK):
            step(j, j % NIB, j % NRB)

        scatter_wait((NCHUNK - 1) % NIB, (NCHUNK - 1) % NRB)

        plsc.subcore_barrier()

        def wcopy(k, carry):
            off = (s + NS * k) * UNIT
            pltpu.sync_copy(agg_sh.at[pl.ds(off, UNIT)],
                            out_hbm.at[c, pl.ds(off, UNIT)])
            return carry

        lax.fori_loop(0, nu, wcopy, 0)

    return seg_kernel(m, edges4)


def _mm_body(x_ref, w_ref, o_ref):
    o_ref[...] = jnp.dot(x_ref[...], w_ref[...],
                         preferred_element_type=jnp.float32)


def _message_mm(x, w):
    return pl.pallas_call(
        _mm_body,
        grid=(N // BLK,),
        in_specs=[
            pl.BlockSpec((BLK, D), lambda i: (i, 0)),
            pl.BlockSpec((D, D), lambda i: (0, 0)),
        ],
        out_specs=pl.BlockSpec((BLK, D), lambda i: (i, 0)),
        out_shape=jax.ShapeDtypeStruct((N, D), jnp.float32),
    )(x, w)


def _gru_body(p_ref, h_ref, wih_ref, whh_ref, bih_ref, bhh_ref, wn_ref,
              hy_ref, mn_ref):
    agg = p_ref[0] + p_ref[1]
    h = h_ref[...]
    gi = jnp.dot(agg, wih_ref[...], preferred_element_type=jnp.float32)
    gi = gi + bih_ref[...]
    gh = jnp.dot(h, whh_ref[...], preferred_element_type=jnp.float32)
    gh = gh + bhh_ref[...]
    r = jax.nn.sigmoid(gi[:, :D] + gh[:, :D])
    z = jax.nn.sigmoid(gi[:, D:2 * D] + gh[:, D:2 * D])
    n = jnp.tanh(gi[:, 2 * D:] + r * gh[:, 2 * D:])
    hy = (1.0 - z) * n + z * h
    hy_ref[...] = hy
    mn_ref[...] = jnp.dot(hy, wn_ref[...], preferred_element_type=jnp.float32)


def _gru_layer(p, h, wih_t, whh_t, bih, bhh, w_next):
    return pl.pallas_call(
        _gru_body,
        grid=(N // BLK,),
        in_specs=[
            pl.BlockSpec((2, BLK, D), lambda i: (0, i, 0)),
            pl.BlockSpec((BLK, D), lambda i: (i, 0)),
            pl.BlockSpec((D, 3 * D), lambda i: (0, 0)),
            pl.BlockSpec((D, 3 * D), lambda i: (0, 0)),
            pl.BlockSpec((1, 3 * D), lambda i: (0, 0)),
            pl.BlockSpec((1, 3 * D), lambda i: (0, 0)),
            pl.BlockSpec((D, D), lambda i: (0, 0)),
        ],
        out_specs=[
            pl.BlockSpec((BLK, D), lambda i: (i, 0)),
            pl.BlockSpec((BLK, D), lambda i: (i, 0)),
        ],
        out_shape=[
            jax.ShapeDtypeStruct((N, D), jnp.float32),
            jax.ShapeDtypeStruct((N, D), jnp.float32),
        ],
    )(p, h, wih_t, whh_t, bih, bhh, w_next)


def kernel(x_encoded, edge_index, mapping_idx, weight, w_ih, w_hh, b_ih, b_hh):
    del mapping_idx
    src2 = jnp.pad(edge_index[0].reshape(NW, EPW), ((0, 0), (0, EPAD)),
                   constant_values=0)
    dst2 = jnp.pad(edge_index[1].reshape(NW, EPW), ((0, 0), (0, EPAD)),
                   constant_values=N)
    edges4 = jnp.stack([src2.reshape(NW, NCHUNK, CHUNK),
                        dst2.reshape(NW, NCHUNK, CHUNK)], axis=2)
    wih_t = w_ih.T
    whh_t = w_hh.T
    bih = b_ih.reshape(1, 3 * D)
    bhh = b_hh.reshape(1, 3 * D)

    h = x_encoded
    m = _message_mm(x_encoded, weight[0])
    for i in range(L):
        p = _segment_sum_partials(m, edges4)
        h, m = _gru_layer(p, h, wih_t, whh_t, bih, bhh, weight[(i + 1) % L])
    return h


# speedup vs baseline: 1.7217x; 1.0338x over previous
import functools
import math

import jax
import jax.numpy as jnp
from jax import lax
from jax.experimental import pallas as pl
from jax.experimental.pallas import tpu as pltpu
from jax.experimental.pallas import tpu_sc as plsc

N = 10000
E = 320000
D = 128
L = 3

NC = 2
NS = 16
NW = NC * NS
EPW = E // NW
CHUNK = 80
NCHUNK = -(-EPW // CHUNK)
EPAD = NCHUNK * CHUNK - EPW
NDUMMY = 8 if EPAD else 0
NRB = 3
NIB = 6
UNIT = 80
NUNITS = N // UNIT

BLK = 1000


def _segment_sum_partials(m, edges4):
    mesh = plsc.VectorSubcoreMesh(core_axis_name="c", subcore_axis_name="s")

    @functools.partial(
        pl.kernel,
        mesh=mesh,
        out_type=jax.ShapeDtypeStruct((NC, N, D), jnp.float32),
        scratch_types=[
            pltpu.VMEM((NIB, 2, CHUNK), jnp.int32),
            pltpu.VMEM((NRB, CHUNK, D), jnp.float32),
            pltpu.VMEM_SHARED((N + NDUMMY, D), jnp.float32),
            [pltpu.SemaphoreType.DMA] * NIB,
            [pltpu.SemaphoreType.DMA] * NRB,
            [pltpu.SemaphoreType.DMA] * NRB,
        ],
    )
    def seg_kernel(m_hbm, edges_hbm, out_hbm,
                   idx_v, rows_v, agg_sh, isems, gsems, ssems):
        c = lax.axis_index("c")
        s = lax.axis_index("s")
        wid = c * NS + s
        nu = jnp.where(s < NUNITS - NS * (NUNITS // NS), NUNITS // NS + 1,
                       NUNITS // NS)

        def zfill(i, carry):
            for g in range(D // 16):
                rows_v[0, i, pl.ds(g * 16, 16)] = jnp.zeros((16,), jnp.float32)
            return carry

        lax.fori_loop(0, UNIT, zfill, 0)

        def zcopy(k, carry):
            pltpu.sync_copy(rows_v.at[0, pl.ds(0, UNIT)],
                            agg_sh.at[pl.ds((s + NS * k) * UNIT, UNIT)])
            return carry

        lax.fori_loop(0, nu, zcopy, 0)

        plsc.subcore_barrier()

        def idx_load(j, q):
            pltpu.async_copy(edges_hbm.at[wid, j], idx_v.at[q], isems[q])

        def idx_wait(j, q):
            pltpu.make_async_copy(edges_hbm.at[wid, j], idx_v.at[q],
                                  isems[q]).wait()

        def gather(q, r):
            pltpu.async_copy(m_hbm.at[idx_v.at[q, 0]], rows_v.at[r],
                             gsems[r])

        def gather_wait(q, r):
            pltpu.make_async_copy(m_hbm.at[idx_v.at[q, 0]], rows_v.at[r],
                                  gsems[r]).wait()

        def scatter(q, r):
            pltpu.async_copy(rows_v.at[r], agg_sh.at[idx_v.at[q, 1]],
                             ssems[r], add=True)

        def scatter_wait(q, r):
            pltpu.make_async_copy(rows_v.at[r], agg_sh.at[idx_v.at[q, 1]],
                                  ssems[r]).wait()

        def step(j, q, r, first=False):
            gather_wait(q, r)
            scatter(q, r)
            if not first:
                scatter_wait((q + NIB - 1) % NIB, (r + NRB - 1) % NRB)

            @pl.when(j + NRB - 1 < NCHUNK)
            def _():
                idx_wait(j + NRB - 1, (q + NRB - 1) % NIB)
                gather((q + NRB - 1) % NIB, (r + NRB - 1) % NRB)

            @pl.when(j + NIB - 2 < NCHUNK)
            def _():
                idx_load(j + NIB - 2, (q + NIB - 2) % NIB)

        for q in range(NIB - 2):
            idx_load(q, q)
        for t in range(NRB - 1):
            idx_wait(t, t)
            gather(t, t)

        JS = NRB - 1
        UNROLL = NRB * NIB // math.gcd(NRB, NIB)
        for j in range(JS):
            step(j, j % NIB, j % NRB, first=(j == 0))

        def main_body(k, carry):
            j0 = JS + UNROLL * k
            for i in range(UNROLL):
                step(j0 + i, (JS + i) % NIB, (JS + i) % NRB)
            return carry

        n_main = (NCHUNK - JS) // UNROLL
        lax.fori_loop(0, n_main, main_body, 0)
        for j in range(JS + UNROLL * n_main, NCHUNK):
            step(j, j % NIB, j % NRB)

        scatter_wait((NCHUNK - 1) % NIB, (NCHUNK - 1) % NRB)

        plsc.subcore_barrier()

        def wcopy(k, carry):
            off = (s + NS * k) * UNIT
            pltpu.sync_copy(agg_sh.at[pl.ds(off, UNIT)],
                            out_hbm.at[c, pl.ds(off, UNIT)])
            return carry

        lax.fori_loop(0, nu, wcopy, 0)

    return seg_kernel(m, edges4)


def _mm_body(x_ref, w_ref, o_ref):
    o_ref[...] = jnp.dot(x_ref[...], w_ref[...],
                         preferred_element_type=jnp.float32)


def _message_mm(x, w):
    return pl.pallas_call(
        _mm_body,
        grid=(N // BLK,),
        in_specs=[
            pl.BlockSpec((BLK, D), lambda i: (i, 0)),
            pl.BlockSpec((D, D), lambda i: (0, 0)),
        ],
        out_specs=pl.BlockSpec((BLK, D), lambda i: (i, 0)),
        out_shape=jax.ShapeDtypeStruct((N, D), jnp.float32),
    )(x, w)


def _gru_body(p_ref, h_ref, wih_ref, whh_ref, bih_ref, bhh_ref, wn_ref,
              hy_ref, mn_ref):
    agg = p_ref[0] + p_ref[1]
    h = h_ref[...]
    gi = jnp.dot(agg, wih_ref[...], preferred_element_type=jnp.float32)
    gi = gi + bih_ref[...]
    gh = jnp.dot(h, whh_ref[...], preferred_element_type=jnp.float32)
    gh = gh + bhh_ref[...]
    r = jax.nn.sigmoid(gi[:, :D] + gh[:, :D])
    z = jax.nn.sigmoid(gi[:, D:2 * D] + gh[:, D:2 * D])
    n = jnp.tanh(gi[:, 2 * D:] + r * gh[:, 2 * D:])
    hy = (1.0 - z) * n + z * h
    hy_ref[...] = hy
    mn_ref[...] = jnp.dot(hy, wn_ref[...], preferred_element_type=jnp.float32)


def _gru_layer(p, h, wih_t, whh_t, bih, bhh, w_next):
    return pl.pallas_call(
        _gru_body,
        grid=(N // BLK,),
        in_specs=[
            pl.BlockSpec((2, BLK, D), lambda i: (0, i, 0)),
            pl.BlockSpec((BLK, D), lambda i: (i, 0)),
            pl.BlockSpec((D, 3 * D), lambda i: (0, 0)),
            pl.BlockSpec((D, 3 * D), lambda i: (0, 0)),
            pl.BlockSpec((1, 3 * D), lambda i: (0, 0)),
            pl.BlockSpec((1, 3 * D), lambda i: (0, 0)),
            pl.BlockSpec((D, D), lambda i: (0, 0)),
        ],
        out_specs=[
            pl.BlockSpec((BLK, D), lambda i: (i, 0)),
            pl.BlockSpec((BLK, D), lambda i: (i, 0)),
        ],
        out_shape=[
            jax.ShapeDtypeStruct((N, D), jnp.float32),
            jax.ShapeDtypeStruct((N, D), jnp.float32),
        ],
    )(p, h, wih_t, whh_t, bih, bhh, w_next)


def kernel(x_encoded, edge_index, mapping_idx, weight, w_ih, w_hh, b_ih, b_hh):
    del mapping_idx
    src2 = jnp.pad(edge_index[0].reshape(NW, EPW), ((0, 0), (0, EPAD)),
                   constant_values=0)
    dst2 = jnp.pad(edge_index[1].reshape(NW, EPW), ((0, 0), (0, EPAD)),
                   constant_values=N)
    edges4 = jnp.stack([src2.reshape(NW, NCHUNK, CHUNK),
                        dst2.reshape(NW, NCHUNK, CHUNK)], axis=2)
    wih_t = w_ih.T
    whh_t = w_hh.T
    bih = b_ih.reshape(1, 3 * D)
    bhh = b_hh.reshape(1, 3 * D)

    h = x_encoded
    m = _message_mm(x_encoded, weight[0])
    for i in range(L):
        p = _segment_sum_partials(m, edges4)
        h, m = _gru_layer(p, h, wih_t, whh_t, bih, bhh, weight[(i + 1) % L])
    return h


# speedup vs baseline: 1.8734x; 1.0881x over previous
import functools
import math

import jax
import jax.numpy as jnp
from jax import lax
from jax.experimental import pallas as pl
from jax.experimental.pallas import tpu as pltpu
from jax.experimental.pallas import tpu_sc as plsc

N = 10000
E = 320000
D = 128
L = 3

NC = 2
NS = 16
NW = NC * NS
EPW = E // NW
CHUNK = 80
NCHUNK = EPW // CHUNK
NRB = 3
NIB = 6
UNIT = 80
NUNITS = N // UNIT
assert EPW % CHUNK == 0 and N % UNIT == 0 and UNIT <= CHUNK

BLK = 1000


def _segment_sum_partials(h, src, dst):
    mesh = plsc.VectorSubcoreMesh(core_axis_name="c", subcore_axis_name="s")

    @functools.partial(
        pl.kernel,
        mesh=mesh,
        out_type=jax.ShapeDtypeStruct((NC, N, D), jnp.float32),
        scratch_types=[
            pltpu.VMEM((NIB, 2, CHUNK), jnp.int32),
            pltpu.VMEM((NRB, CHUNK, D), jnp.float32),
            pltpu.VMEM_SHARED((N, D), jnp.float32),
            [pltpu.SemaphoreType.DMA] * NIB,
            [pltpu.SemaphoreType.DMA] * NRB,
            [pltpu.SemaphoreType.DMA] * NRB,
        ],
    )
    def seg_kernel(h_hbm, src_hbm, dst_hbm, out_hbm,
                   idx_v, rows_v, agg_sh, isems, gsems, ssems):
        c = lax.axis_index("c")
        s = lax.axis_index("s")
        base = (c * NS + s) * EPW
        nu = jnp.where(s < NUNITS - NS * (NUNITS // NS), NUNITS // NS + 1,
                       NUNITS // NS)

        def zfill(i, carry):
            for g in range(D // 16):
                rows_v[0, i, pl.ds(g * 16, 16)] = jnp.zeros((16,), jnp.float32)
            return carry

        lax.fori_loop(0, UNIT, zfill, 0)

        def zcopy(k, carry):
            pltpu.sync_copy(rows_v.at[0, pl.ds(0, UNIT)],
                            agg_sh.at[pl.ds((s + NS * k) * UNIT, UNIT)])
            return carry

        lax.fori_loop(0, nu, zcopy, 0)

        plsc.subcore_barrier()

        def idx_load(j, q):
            off = base + j * CHUNK
            pltpu.async_copy(src_hbm.at[pl.ds(off, CHUNK)],
                             idx_v.at[q, 0], isems[q])
            pltpu.async_copy(dst_hbm.at[pl.ds(off, CHUNK)],
                             idx_v.at[q, 1], isems[q])

        def idx_wait(j, q):
            off = base + j * CHUNK
            pltpu.make_async_copy(src_hbm.at[pl.ds(off, CHUNK)],
                                  idx_v.at[q, 0], isems[q]).wait()
            pltpu.make_async_copy(dst_hbm.at[pl.ds(off, CHUNK)],
                                  idx_v.at[q, 1], isems[q]).wait()

        def gather(q, r):
            pltpu.async_copy(h_hbm.at[idx_v.at[q, 0]], rows_v.at[r],
                             gsems[r])

        def gather_wait(q, r):
            pltpu.make_async_copy(h_hbm.at[idx_v.at[q, 0]], rows_v.at[r],
                                  gsems[r]).wait()

        def scatter(q, r):
            pltpu.async_copy(rows_v.at[r], agg_sh.at[idx_v.at[q, 1]],
                             ssems[r], add=True)

        def scatter_wait(q, r):
            pltpu.make_async_copy(rows_v.at[r], agg_sh.at[idx_v.at[q, 1]],
                                  ssems[r]).wait()

        def step(j, q, r, first=False):
            gather_wait(q, r)
            scatter(q, r)
            if not first:
                scatter_wait((q + NIB - 1) % NIB, (r + NRB - 1) % NRB)

            @pl.when(j + NRB - 1 < NCHUNK)
            def _():
                idx_wait(j + NRB - 1, (q + NRB - 1) % NIB)
                gather((q + NRB - 1) % NIB, (r + NRB - 1) % NRB)

            @pl.when(j + NIB - 2 < NCHUNK)
            def _():
                idx_load(j + NIB - 2, (q + NIB - 2) % NIB)

        for q in range(NIB - 2):
            idx_load(q, q)
        for t in range(NRB - 1):
            idx_wait(t, t)
            gather(t, t)

        JS = NRB - 1
        UNROLL = NRB * NIB // math.gcd(NRB, NIB)
        for j in range(JS):
            step(j, j % NIB, j % NRB, first=(j == 0))

        def main_body(k, carry):
            j0 = JS + UNROLL * k
            for i in range(UNROLL):
                step(j0 + i, (JS + i) % NIB, (JS + i) % NRB)
            return carry

        n_main = (NCHUNK - JS) // UNROLL
        lax.fori_loop(0, n_main, main_body, 0)
        for j in range(JS + UNROLL * n_main, NCHUNK):
            step(j, j % NIB, j % NRB)

        scatter_wait((NCHUNK - 1) % NIB, (NCHUNK - 1) % NRB)

        plsc.subcore_barrier()

        def wcopy(k, carry):
            off = (s + NS * k) * UNIT
            pltpu.sync_copy(agg_sh.at[pl.ds(off, UNIT)],
                            out_hbm.at[c, pl.ds(off, UNIT)])
            return carry

        lax.fori_loop(0, nu, wcopy, 0)

    return seg_kernel(h, src, dst)


def _gru_body(p_ref, h_ref, w_ref, wih_ref, whh_ref, bih_ref, bhh_ref,
              hy_ref):
    agg = jnp.dot(p_ref[0] + p_ref[1], w_ref[...],
                  preferred_element_type=jnp.float32)
    h = h_ref[...]
    cdims = (((1,), (1,)), ((), ()))
    gi = lax.dot_general(agg, wih_ref[...], cdims,
                         preferred_element_type=jnp.float32) + bih_ref[...]
    gh = lax.dot_general(h, whh_ref[...], cdims,
                         preferred_element_type=jnp.float32) + bhh_ref[...]
    r = jax.nn.sigmoid(gi[:, :D] + gh[:, :D])
    z = jax.nn.sigmoid(gi[:, D:2 * D] + gh[:, D:2 * D])
    n = jnp.tanh(gi[:, 2 * D:] + r * gh[:, 2 * D:])
    hy_ref[...] = (1.0 - z) * n + z * h


def _gru_layer(p, h, w, w_ih, w_hh, bih, bhh):
    return pl.pallas_call(
        _gru_body,
        grid=(N // BLK,),
        in_specs=[
            pl.BlockSpec((2, BLK, D), lambda i: (0, i, 0)),
            pl.BlockSpec((BLK, D), lambda i: (i, 0)),
            pl.BlockSpec((D, D), lambda i: (0, 0)),
            pl.BlockSpec((3 * D, D), lambda i: (0, 0)),
            pl.BlockSpec((3 * D, D), lambda i: (0, 0)),
            pl.BlockSpec((1, 3 * D), lambda i: (0, 0)),
            pl.BlockSpec((1, 3 * D), lambda i: (0, 0)),
        ],
        out_specs=pl.BlockSpec((BLK, D), lambda i: (i, 0)),
        out_shape=jax.ShapeDtypeStruct((N, D), jnp.float32),
    )(p, h, w, w_ih, w_hh, bih, bhh)


def kernel(x_encoded, edge_index, mapping_idx, weight, w_ih, w_hh, b_ih, b_hh):
    del mapping_idx
    bih = b_ih.reshape(1, 3 * D)
    bhh = b_hh.reshape(1, 3 * D)
    src = edge_index[0]
    dst = edge_index[1]

    h = x_encoded
    for i in range(L):
        p = _segment_sum_partials(h, src, dst)
        h = _gru_layer(p, h, weight[i], w_ih, w_hh, bih, bhh)
    return h
